# baseline (speedup 1.0000x reference)
import functools

import jax
import jax.numpy as jnp
from jax import lax
from jax.experimental import pallas as pl
from jax.experimental.pallas import tpu as pltpu
from jax.experimental.pallas import tpu_sc as plsc

B = 16384
EMB = 64
HID = 192
EPS = 1e-5
NC, NS = 2, 16
NW = NC * NS
BPW = B // NW
CHUNK = 128
NCHUNK = BPW // CHUNK


def _sc_gather(i1, i2, i3, i4, t1, t2, t3, t4):
    mesh = plsc.VectorSubcoreMesh(core_axis_name="c", subcore_axis_name="s")
    out_type = [jax.ShapeDtypeStruct((B, EMB), jnp.float32) for _ in range(4)]
    scratch = (
        [pltpu.VMEM((BPW,), jnp.int32) for _ in range(4)]
        + [pltpu.VMEM((BPW, EMB), jnp.float32) for _ in range(2)]
        + [pltpu.SemaphoreType.DMA]
    )

    @functools.partial(pl.kernel, mesh=mesh, out_type=out_type,
                       scratch_types=scratch,
                       compiler_params=pltpu.CompilerParams(
                           use_tc_tiling_on_sc=False))
    def k(i1r, i2r, i3r, i4r, t1r, t2r, t3r, t4r,
          o1r, o2r, o3r, o4r, iv1, iv2, iv3, iv4, rows_a, rows_b, sem):
        wid = lax.axis_index("s") * NC + lax.axis_index("c")
        base = wid * BPW
        idx_refs = [iv1, iv2, iv3, iv4]
        in_refs = [i1r, i2r, i3r, i4r]
        tab_refs = [t1r, t2r, t3r, t4r]
        out_refs = [o1r, o2r, o3r, o4r]
        rows = [rows_a, rows_b]
        for t in range(4):
            pltpu.sync_copy(in_refs[t].at[pl.ds(base, BPW)], idx_refs[t])
        copies = [None, None, None, None]
        for t in range(4):
            buf = rows[t % 2]
            cps = []
            for j in range(NCHUNK):
                cps.append(pltpu.async_copy(
                    tab_refs[t].at[idx_refs[t].at[pl.ds(j * CHUNK, CHUNK)]],
                    buf.at[pl.ds(j * CHUNK, CHUNK), :], sem))
            copies[t] = cps
            if t >= 1:
                for cp in copies[t - 1]:
                    cp.wait()
                pltpu.sync_copy(rows[(t - 1) % 2],
                                out_refs[t - 1].at[pl.ds(base, BPW)])
        for cp in copies[3]:
            cp.wait()
        pltpu.sync_copy(rows[3 % 2], out_refs[3].at[pl.ds(base, BPW)])

    return k(i1, i2, i3, i4, t1, t2, t3, t4)


BCHUNK = 1024
NBCHUNK = B // BCHUNK


def _stats_body(e1, e2, e3, e4, gamma, beta, g, v, bias,
                ws_out, b2_out, acc):
    step = pl.program_id(0)

    @pl.when(step == 0)
    def _init():
        acc[...] = jnp.zeros_like(acc)

    x = jnp.concatenate([e1[...], e2[...], e3[...], e4[...]], axis=1)
    acc[0:1, :] += jnp.sum(x, axis=0, keepdims=True)
    acc[1:2, :] += jnp.sum(x * x, axis=0, keepdims=True)

    @pl.when(step == NBCHUNK - 1)
    def _finalize():
        mean = acc[0:1, :] / B
        var = acc[1:2, :] / B - mean * mean
        s = gamma[...][None, :] / jnp.sqrt(var + EPS)
        shift = beta[...][None, :] - mean * s
        vv = v[...]
        v_norm = jnp.sqrt(jnp.sum(vv * vv, axis=1, keepdims=True))
        W = (g[...][:, None] / v_norm) * vv
        ws_out[...] = W * s
        b2 = bias[...] + lax.dot_general(
            W, shift[0], (((1,), (0,)), ((), ())),
            preferred_element_type=jnp.float32)
        b2_out[...] = b2[None, :]


def _matmul_body(e1, e2, e3, e4, ws, b2, out):
    x = jnp.concatenate([e1[...], e2[...], e3[...], e4[...]], axis=1)
    y = lax.dot_general(x, ws[...], (((1,), (1,)), ((), ())),
                        preferred_element_type=jnp.float32)
    out[...] = jax.nn.sigmoid(y + b2[...])


def _tc_stage(e1, e2, e3, e4, bn_gamma, bn_beta, wn_g, wn_v, bias):
    CAT = 4 * EMB
    echunk = pl.BlockSpec((BCHUNK, EMB), lambda i: (i, 0))
    full = lambda shape: pl.BlockSpec(shape, lambda i: tuple(0 for _ in shape))
    ws, b2 = pl.pallas_call(
        _stats_body,
        grid=(NBCHUNK,),
        in_specs=[echunk] * 4 + [full((CAT,)), full((CAT,)), full((HID,)),
                                 full((HID, CAT)), full((HID,))],
        out_specs=[full((HID, CAT)), full((1, HID))],
        out_shape=[jax.ShapeDtypeStruct((HID, CAT), jnp.float32),
                   jax.ShapeDtypeStruct((1, HID), jnp.float32)],
        scratch_shapes=[pltpu.VMEM((2, CAT), jnp.float32)],
    )(e1, e2, e3, e4, bn_gamma, bn_beta, wn_g, wn_v, bias)
    out = pl.pallas_call(
        _matmul_body,
        grid=(NBCHUNK,),
        in_specs=[echunk] * 4 + [full((HID, CAT)), full((1, HID))],
        out_specs=pl.BlockSpec((BCHUNK, HID), lambda i: (i, 0)),
        out_shape=jax.ShapeDtypeStruct((B, HID), jnp.float32),
    )(e1, e2, e3, e4, ws, b2)
    return out


def kernel(last_test, last_question, last_tag, last_qclass,
           emb_test, emb_question, emb_tag, emb_qclass,
           bn_gamma, bn_beta, wn_g, wn_v, bias):
    i1 = last_test.astype(jnp.int32)
    i2 = last_question.astype(jnp.int32)
    i3 = last_tag.astype(jnp.int32)
    i4 = last_qclass.astype(jnp.int32)
    e1, e2, e3, e4 = _sc_gather(i1, i2, i3, i4,
                                emb_test, emb_question, emb_tag, emb_qclass)
    return _tc_stage(e1, e2, e3, e4, bn_gamma, bn_beta, wn_g, wn_v, bias)
